# baseline (speedup 1.0000x reference)
import functools

import jax
import jax.numpy as jnp
from jax import lax
from jax.experimental import pallas as pl
from jax.experimental.pallas import tpu as pltpu
from jax.experimental.pallas import tpu_sc as plsc

N = 16384
C = 1000
NT = 16
B = N // NT
NCHUNK = 8
CHUNK = B // NCHUNK
L = 16


@functools.partial(
    pl.kernel,
    mesh=plsc.VectorSubcoreMesh(core_axis_name="c", subcore_axis_name="s",
                                num_cores=1),
    out_type=jax.ShapeDtypeStruct((NT, L), jnp.float32),
    scratch_types=[
        pltpu.VMEM((NCHUNK, CHUNK), jnp.int32),
        pltpu.VMEM((NCHUNK, CHUNK), jnp.float32),
        pltpu.VMEM((B,), jnp.int32),
        pltpu.VMEM((B,), jnp.float32),
        pltpu.VMEM((L,), jnp.float32),
        pltpu.SemaphoreType.DMA,
    ],
)
def _gan_loss_sc(prob_hbm, target_hbm, reward_hbm, out_hbm,
                 idx_v, vals_v, tgt_v, rew_v, part_v, sem):
    sid = lax.axis_index("s")
    base = sid * B

    pltpu.sync_copy(target_hbm.at[pl.ds(base, B)], tgt_v)
    pltpu.sync_copy(reward_hbm.at[pl.ds(base, B)], rew_v)

    for r in range(NCHUNK):
        for k in range(CHUNK // L):
            off = r * CHUNK + k * L
            t = tgt_v[pl.ds(off, L)]
            rows = (base + off) + lax.iota(jnp.int32, L)
            idx_v[r, pl.ds(k * L, L)] = rows * C + t

    copies = [
        pltpu.async_copy(prob_hbm.at[idx_v.at[r]], vals_v.at[r], sem)
        for r in range(NCHUNK)
    ]
    for cp in copies:
        cp.wait()

    acc = jnp.zeros((L,), jnp.float32)
    for r in range(NCHUNK):
        for k in range(CHUNK // L):
            off = r * CHUNK + k * L
            acc = acc + vals_v[r, pl.ds(k * L, L)] * rew_v[pl.ds(off, L)]

    part_v[...] = acc
    pltpu.sync_copy(part_v, out_hbm.at[sid])


def _reduce_tc_body(parts_ref, out_ref):
    out_ref[0, 0] = -jnp.sum(parts_ref[...])


_reduce_tc = pl.pallas_call(
    _reduce_tc_body,
    out_shape=jax.ShapeDtypeStruct((1, 1), jnp.float32),
    in_specs=[pl.BlockSpec(memory_space=pltpu.VMEM)],
    out_specs=pl.BlockSpec(memory_space=pltpu.SMEM),
)


def kernel(prob, target, reward):
    parts = _gan_loss_sc(prob.reshape(-1), target, reward)
    return jnp.reshape(_reduce_tc(parts), ())


# speedup vs baseline: 1.5386x; 1.5386x over previous
import jax
import jax.numpy as jnp
from jax.experimental import pallas as pl
from jax.experimental.pallas import tpu as pltpu

N = 16384
C = 1000
RB = 512
G = N // RB


def _body(tgt_ref, rew_ref, prob_ref, out_ref):
    i = pl.program_id(0)
    tgt = tgt_ref[0, 0, :]
    rew = rew_ref[0, 0, :]
    pb = prob_ref[...]
    cols = jax.lax.broadcasted_iota(jnp.int32, (RB, C), 1)
    sel = jnp.where(cols == tgt[:, None], pb, 0.0)
    part = jnp.sum(jnp.sum(sel, axis=1) * rew)

    @pl.when(i == 0)
    def _():
        out_ref[0, 0] = 0.0

    out_ref[0, 0] = out_ref[0, 0] - part


_tc_loss = pl.pallas_call(
    _body,
    grid=(G,),
    in_specs=[
        pl.BlockSpec((1, 1, RB), lambda i: (i, 0, 0)),
        pl.BlockSpec((1, 1, RB), lambda i: (i, 0, 0)),
        pl.BlockSpec((RB, C), lambda i: (i, 0)),
    ],
    out_specs=pl.BlockSpec(memory_space=pltpu.SMEM),
    out_shape=jax.ShapeDtypeStruct((1, 1), jnp.float32),
)


def kernel(prob, target, reward):
    t3 = target.reshape(G, 1, RB)
    r3 = reward.reshape(G, 1, RB)
    return jnp.reshape(_tc_loss(t3, r3, prob), ())


# speedup vs baseline: 1.7361x; 1.1283x over previous
import jax
import jax.numpy as jnp
from jax.experimental import pallas as pl
from jax.experimental.pallas import tpu as pltpu

N = 16384
C = 1000
RB = 1024
G = N // RB


def _body(tgt_ref, rew_ref, prob_ref, out_ref):
    i = pl.program_id(0)
    tgt = tgt_ref[...]
    rew = rew_ref[...]
    pb = prob_ref[...]
    cols = jax.lax.broadcasted_iota(jnp.int32, (RB, C), 1)
    sel = jnp.where(cols == tgt[:, None], pb, 0.0)
    part = jnp.sum(jnp.sum(sel, axis=1) * rew)

    @pl.when(i == 0)
    def _():
        out_ref[0, 0] = 0.0

    out_ref[0, 0] = out_ref[0, 0] - part


_tc_loss = pl.pallas_call(
    _body,
    grid=(G,),
    in_specs=[
        pl.BlockSpec((RB,), lambda i: (i,)),
        pl.BlockSpec((RB,), lambda i: (i,)),
        pl.BlockSpec((RB, C), lambda i: (i, 0)),
    ],
    out_specs=pl.BlockSpec(memory_space=pltpu.SMEM),
    out_shape=jax.ShapeDtypeStruct((1, 1), jnp.float32),
)


def kernel(prob, target, reward):
    return jnp.reshape(_tc_loss(target, reward, prob), ())


# speedup vs baseline: 6.4654x; 3.7242x over previous
import functools

import jax
import jax.numpy as jnp
from jax import lax
from jax.experimental import pallas as pl
from jax.experimental.pallas import tpu as pltpu
from jax.experimental.pallas import tpu_sc as plsc

N = 16384
C = 1000
NT = 16
B = N // NT
NCHUNK = 8
CHUNK = B // NCHUNK
L = 16


@functools.partial(
    pl.kernel,
    mesh=plsc.VectorSubcoreMesh(core_axis_name="c", subcore_axis_name="s",
                                num_cores=1),
    out_type=jax.ShapeDtypeStruct((NT, L), jnp.float32),
    scratch_types=[
        pltpu.VMEM((NCHUNK, CHUNK), jnp.int32),
        pltpu.VMEM((NCHUNK, CHUNK), jnp.float32),
        pltpu.VMEM((B,), jnp.int32),
        pltpu.VMEM((B,), jnp.float32),
        pltpu.VMEM((L,), jnp.float32),
        pltpu.SemaphoreType.DMA,
    ],
)
def _gan_loss_sc(prob_hbm, target_hbm, reward_hbm, out_hbm,
                 idx_v, vals_v, tgt_v, rew_v, part_v, sem):
    sid = lax.axis_index("s")
    base = sid * B

    pltpu.sync_copy(target_hbm.at[pl.ds(base, B)], tgt_v)
    pltpu.sync_copy(reward_hbm.at[pl.ds(base, B)], rew_v)

    for r in range(NCHUNK):
        for k in range(CHUNK // L):
            off = r * CHUNK + k * L
            t = tgt_v[pl.ds(off, L)]
            n = (base + off) + lax.iota(jnp.int32, L)
            idx = (((t >> 3) << 17) + ((n >> 7) << 10)
                   + ((t & 7) << 7) + (n & 127))
            idx_v[r, pl.ds(k * L, L)] = idx

    copies = [
        pltpu.async_copy(prob_hbm.at[idx_v.at[r]], vals_v.at[r], sem)
        for r in range(NCHUNK)
    ]
    for cp in copies:
        cp.wait()

    acc = jnp.zeros((L,), jnp.float32)
    for r in range(NCHUNK):
        for k in range(CHUNK // L):
            off = r * CHUNK + k * L
            acc = acc + vals_v[r, pl.ds(k * L, L)] * rew_v[pl.ds(off, L)]

    part_v[...] = acc
    pltpu.sync_copy(part_v, out_hbm.at[sid])


def _reduce_tc_body(parts_ref, out_ref):
    out_ref[0, 0] = -jnp.sum(parts_ref[...])


_reduce_tc = pl.pallas_call(
    _reduce_tc_body,
    out_shape=jax.ShapeDtypeStruct((1, 1), jnp.float32),
    in_specs=[pl.BlockSpec(memory_space=pltpu.VMEM)],
    out_specs=pl.BlockSpec(memory_space=pltpu.SMEM),
)


def kernel(prob, target, reward):
    flat = (prob.T.reshape(C // 8, 8, N // 128, 128)
            .transpose(0, 2, 1, 3)
            .reshape(N * C))
    parts = _gan_loss_sc(flat, target, reward)
    return jnp.reshape(_reduce_tc(parts), ())
